# Initial kernel scaffold; baseline (speedup 1.0000x reference)
import jax
import jax.numpy as jnp
from jax.experimental import pallas as pl


def kernel(user, traj, user_table, loc_table):
    raise NotImplementedError("write your pallas kernel here")


# speedup vs baseline: 4.0087x; 4.0087x over previous
import functools

import jax
import jax.numpy as jnp
from jax import lax
from jax.experimental import pallas as pl
from jax.experimental.pallas import tpu as pltpu
from jax.experimental.pallas import tpu_sc as plsc

_INFO = plsc.get_sparse_core_info()
_NC, _NS = _INFO.num_cores, _INFO.num_subcores
_NW = _NC * _NS

_G = 128
_JB = 4
_CH = _G * _JB
_D = 64

_N_TRAJ = 16384 * 200
_N_USER = 16384
_TRAJ_CHUNKS = _N_TRAJ // (_NW * _CH)
_USER_CHUNKS = _N_USER // (_NW * _CH)


def _sc_body(user_hbm, traj_hbm, utab_hbm, ltab_hbm, uout_hbm, tout_hbm,
             idx_v, rows_v, sem):
  wid = lax.axis_index("s") * _NC + lax.axis_index("c")

  def do_chunk(idx2d_hbm, tab_hbm, out_hbm, base_row, base_out):
    pltpu.sync_copy(idx2d_hbm.at[pl.ds(base_row, _JB)], idx_v)
    copies = []
    for j in range(_JB):
      copies.append(
          pltpu.async_copy(tab_hbm.at[idx_v.at[j]],
                           rows_v.at[pl.ds(j * _G, _G)], sem))
    for c in copies:
      c.wait()
    pltpu.sync_copy(rows_v, out_hbm.at[pl.ds(base_out, _CH)])

  traj_base = wid * _TRAJ_CHUNKS

  @pl.loop(0, _TRAJ_CHUNKS)
  def _(g):
    do_chunk(traj_hbm, ltab_hbm, tout_hbm,
             (traj_base + g) * _JB, (traj_base + g) * _CH)

  user_base = wid * _USER_CHUNKS

  @pl.loop(0, _USER_CHUNKS)
  def _(g):
    do_chunk(user_hbm, utab_hbm, uout_hbm,
             (user_base + g) * _JB, (user_base + g) * _CH)


@jax.jit
def _sc_embed(user2d, traj2d, user_table, loc_table):
  mesh = plsc.VectorSubcoreMesh(core_axis_name="c", subcore_axis_name="s")
  fn = pl.kernel(
      _sc_body,
      out_type=(
          jax.ShapeDtypeStruct((_N_USER, _D), jnp.float32),
          jax.ShapeDtypeStruct((_N_TRAJ, _D), jnp.float32),
      ),
      mesh=mesh,
      compiler_params=pltpu.CompilerParams(use_tc_tiling_on_sc=False),
      scratch_types=[
          pltpu.VMEM((_JB, _G), jnp.int32),
          pltpu.VMEM((_CH, _D), jnp.float32),
          pltpu.SemaphoreType.DMA,
      ],
  )
  return fn(user2d, traj2d, user_table, loc_table)


def kernel(user, traj, user_table, loc_table):
  user2d = user.astype(jnp.int32).reshape(_N_USER // _G, _G)
  traj2d = traj.astype(jnp.int32).reshape(_N_TRAJ // _G, _G)
  uout, tout = _sc_embed(user2d, traj2d, user_table, loc_table)
  return uout, tout.reshape(16384, 200, _D)


# speedup vs baseline: 4.2940x; 1.0712x over previous
import functools

import jax
import jax.numpy as jnp
from jax import lax
from jax.experimental import pallas as pl
from jax.experimental.pallas import tpu as pltpu
from jax.experimental.pallas import tpu_sc as plsc

_INFO = plsc.get_sparse_core_info()
_NC, _NS = _INFO.num_cores, _INFO.num_subcores
_NW = _NC * _NS

_G = 128
_JB = 4
_CH = _G * _JB
_D = 64

_N_TRAJ = 16384 * 200
_N_USER = 16384
_TRAJ_CHUNKS = _N_TRAJ // (_NW * _CH)
_USER_CHUNKS = _N_USER // (_NW * _CH)


def _sc_body(user_hbm, traj_hbm, utab_hbm, ltab_hbm, uout_hbm, tout_hbm,
             idx_v, rows_v, isem, gsem, osem):
  wid = lax.axis_index("s") * _NC + lax.axis_index("c")
  base = wid * _TRAJ_CHUNKS

  def start_idx(g, b):
    pltpu.async_copy(traj_hbm.at[pl.ds((base + g) * _JB, _JB)],
                     idx_v.at[b], isem.at[b])

  def wait_idx(b):
    pltpu.make_async_copy(traj_hbm.at[pl.ds(0, _JB)], idx_v.at[b],
                          isem.at[b]).wait()

  def start_gathers(b):
    for j in range(_JB):
      pltpu.async_copy(ltab_hbm.at[idx_v.at[b, j]],
                       rows_v.at[b, pl.ds(j * _G, _G)], gsem.at[b])

  def wait_gathers(b):
    for j in range(_JB):
      pltpu.make_async_copy(ltab_hbm.at[pl.ds(0, _G)],
                            rows_v.at[b, pl.ds(j * _G, _G)],
                            gsem.at[b]).wait()

  def start_out(g, b):
    pltpu.async_copy(rows_v.at[b], tout_hbm.at[pl.ds((base + g) * _CH, _CH)],
                     osem.at[b])

  def wait_out(b):
    pltpu.make_async_copy(rows_v.at[b], tout_hbm.at[pl.ds(0, _CH)],
                          osem.at[b]).wait()

  start_idx(0, 0)
  start_idx(1, 1)
  wait_idx(0)
  start_gathers(0)

  @pl.loop(0, _TRAJ_CHUNKS - 1)
  def _(g):
    cb = lax.rem(g, 2)
    nb = lax.rem(g + 1, 2)

    @pl.when(g >= 1)
    def _():
      wait_out(nb)
    wait_idx(nb)
    start_gathers(nb)
    wait_gathers(cb)

    @pl.when(g + 2 < _TRAJ_CHUNKS)
    def _():
      start_idx(g + 2, cb)
    start_out(g, cb)

  last = _TRAJ_CHUNKS - 1
  lb = last % 2
  wait_gathers(lb)
  start_out(last, lb)
  wait_out(1 - lb)
  wait_out(lb)

  ubase = wid * _USER_CHUNKS
  pltpu.sync_copy(user_hbm.at[pl.ds(ubase * _JB, _JB)], idx_v.at[0])
  ucopies = [
      pltpu.async_copy(utab_hbm.at[idx_v.at[0, j]],
                       rows_v.at[0, pl.ds(j * _G, _G)], gsem.at[0])
      for j in range(_JB)
  ]
  for c in ucopies:
    c.wait()
  pltpu.sync_copy(rows_v.at[0], uout_hbm.at[pl.ds(ubase * _CH, _CH)])


@jax.jit
def _sc_embed(user2d, traj2d, user_table, loc_table):
  mesh = plsc.VectorSubcoreMesh(core_axis_name="c", subcore_axis_name="s")
  fn = pl.kernel(
      _sc_body,
      out_type=(
          jax.ShapeDtypeStruct((_N_USER, _D), jnp.float32),
          jax.ShapeDtypeStruct((_N_TRAJ, _D), jnp.float32),
      ),
      mesh=mesh,
      compiler_params=pltpu.CompilerParams(use_tc_tiling_on_sc=False),
      scratch_types=[
          pltpu.VMEM((2, _JB, _G), jnp.int32),
          pltpu.VMEM((2, _CH, _D), jnp.float32),
          pltpu.SemaphoreType.DMA((2,)),
          pltpu.SemaphoreType.DMA((2,)),
          pltpu.SemaphoreType.DMA((2,)),
      ],
  )
  return fn(user2d, traj2d, user_table, loc_table)


def kernel(user, traj, user_table, loc_table):
  user2d = user.astype(jnp.int32).reshape(_N_USER // _G, _G)
  traj2d = traj.astype(jnp.int32).reshape(_N_TRAJ // _G, _G)
  uout, tout = _sc_embed(user2d, traj2d, user_table, loc_table)
  return uout, tout.reshape(16384, 200, _D)


# speedup vs baseline: 7.5847x; 1.7664x over previous
import functools

import jax
import jax.numpy as jnp
from jax import lax
from jax.experimental import pallas as pl
from jax.experimental.pallas import tpu as pltpu
from jax.experimental.pallas import tpu_sc as plsc

_INFO = plsc.get_sparse_core_info()
_NC, _NS = _INFO.num_cores, _INFO.num_subcores
_NW = _NC * _NS

_G = 128
_JB = 4
_CH = _G * _JB
_D = 64

_N_TRAJ = 16384 * 200
_N_USER = 16384
_TRAJ_CHUNKS = _N_TRAJ // (_NW * _CH)
_USER_CHUNKS = _N_USER // (_NW * _CH)


def _traj_body(traj_hbm, ltab_hbm, tout_hbm, idx_v, rows_v, isem, gsem, osem):
  wid = lax.axis_index("s") * _NC + lax.axis_index("c")
  base = wid * _TRAJ_CHUNKS

  def start_idx(g, b):
    pltpu.async_copy(traj_hbm.at[pl.ds((base + g) * _JB, _JB)],
                     idx_v.at[b], isem.at[b])

  def wait_idx(b):
    pltpu.make_async_copy(traj_hbm.at[pl.ds(0, _JB)], idx_v.at[b],
                          isem.at[b]).wait()

  def start_gathers(b):
    for j in range(_JB):
      pltpu.async_copy(ltab_hbm.at[idx_v.at[b, j]],
                       rows_v.at[b, pl.ds(j * _G, _G)], gsem.at[b])

  def wait_gathers(b):
    for j in range(_JB):
      pltpu.make_async_copy(ltab_hbm.at[pl.ds(0, _G)],
                            rows_v.at[b, pl.ds(j * _G, _G)],
                            gsem.at[b]).wait()

  def start_out(g, b):
    pltpu.async_copy(rows_v.at[b],
                     tout_hbm.at[pl.ds((base + g) * _CH, _CH), pl.ds(0, _D)],
                     osem.at[b])

  def wait_out(b):
    pltpu.make_async_copy(rows_v.at[b],
                          tout_hbm.at[pl.ds(0, _CH), pl.ds(0, _D)],
                          osem.at[b]).wait()

  start_idx(0, 0)
  start_idx(1, 1)
  wait_idx(0)
  start_gathers(0)

  @pl.loop(0, _TRAJ_CHUNKS - 1)
  def _(g):
    cb = lax.rem(g, 2)
    nb = lax.rem(g + 1, 2)

    @pl.when(g >= 1)
    def _():
      wait_out(nb)
    wait_idx(nb)
    start_gathers(nb)
    wait_gathers(cb)

    @pl.when(g + 2 < _TRAJ_CHUNKS)
    def _():
      start_idx(g + 2, cb)
    start_out(g, cb)

  last = _TRAJ_CHUNKS - 1
  lb = last % 2
  wait_gathers(lb)
  start_out(last, lb)
  wait_out(1 - lb)
  wait_out(lb)


def _user_body(user_hbm, utab_hbm, uout_hbm, idx_v, rows_v, gsem):
  wid = lax.axis_index("s") * _NC + lax.axis_index("c")
  ubase = wid * _USER_CHUNKS
  pltpu.sync_copy(user_hbm.at[pl.ds(ubase * _JB, _JB)], idx_v)
  copies = [
      pltpu.async_copy(utab_hbm.at[idx_v.at[j]],
                       rows_v.at[pl.ds(j * _G, _G)], gsem)
      for j in range(_JB)
  ]
  for c in copies:
    c.wait()
  pltpu.sync_copy(rows_v,
                  uout_hbm.at[pl.ds(ubase * _CH, _CH), pl.ds(0, _D)])


@jax.jit
def _sc_embed(user2d, traj2d, user_table, loc_table):
  mesh = plsc.VectorSubcoreMesh(core_axis_name="c", subcore_axis_name="s")
  traj_fn = pl.kernel(
      _traj_body,
      out_type=jax.ShapeDtypeStruct((_N_TRAJ, 2 * _D), jnp.float32),
      mesh=mesh,
      compiler_params=pltpu.CompilerParams(use_tc_tiling_on_sc=False),
      scratch_types=[
          pltpu.VMEM((2, _JB, _G), jnp.int32),
          pltpu.VMEM((2, _CH, _D), jnp.float32),
          pltpu.SemaphoreType.DMA((2,)),
          pltpu.SemaphoreType.DMA((2,)),
          pltpu.SemaphoreType.DMA((2,)),
      ],
  )
  user_fn = pl.kernel(
      _user_body,
      out_type=jax.ShapeDtypeStruct((_N_USER, 2 * _D), jnp.float32),
      mesh=mesh,
      compiler_params=pltpu.CompilerParams(use_tc_tiling_on_sc=False),
      scratch_types=[
          pltpu.VMEM((_JB, _G), jnp.int32),
          pltpu.VMEM((_CH, _D), jnp.float32),
          pltpu.SemaphoreType.DMA,
      ],
  )
  tout = traj_fn(traj2d, loc_table)
  uout = user_fn(user2d, user_table)
  return uout, tout


def kernel(user, traj, user_table, loc_table):
  user2d = user.astype(jnp.int32).reshape(_N_USER // _G, _G)
  traj2d = traj.astype(jnp.int32).reshape(_N_TRAJ // _G, _G)
  uout128, tout128 = _sc_embed(user2d, traj2d, user_table, loc_table)
  return (uout128[:, :_D],
          tout128[:, :_D].reshape(16384, 200, _D))
